# baseline (speedup 1.0000x reference)
import jax
import jax.numpy as jnp
from jax import lax
from jax.experimental import pallas as pl
from jax.experimental.pallas import tpu as pltpu

_N = 10000
_H = 128
_M = 64
_BN = 1000
_NB = _N // _BN

_DN_T = (((1,), (1,)), ((), ()))


def _dot_t(a, b):
    return lax.dot_general(a, b, _DN_T, preferred_element_type=jnp.float32)


def _dot(a, b):
    return jnp.dot(a, b, preferred_element_type=jnp.float32)


def _gru(mem, xm, wih, whh_t, bih, bhh):
    gi_r = _dot_t(xm, wih[0:_M, :]) + bih[0:_M]
    gi_z = _dot_t(xm, wih[_M:2 * _M, :]) + bih[_M:2 * _M]
    gi_n = _dot_t(xm, wih[2 * _M:, :]) + bih[2 * _M:]
    gh = _dot(mem, whh_t[...]) + bhh[...]
    r = jax.nn.sigmoid(gi_r + gh[:, 0:_M])
    z = jax.nn.sigmoid(gi_z + gh[:, _M:2 * _M])
    n = jnp.tanh(gi_n + r * gh[:, 2 * _M:])
    return (1.0 - z) * n + z * mem


def _mlp(hb, mem, wm1_t, bm1, wm2, bm2, wa, ba):
    c = _dot(mem, wm1_t[_H:, :]) + bm1[...]
    u = jnp.maximum(_dot(hb, wm1_t[0:_H, :]) + c, 0.0)
    msg = _dot_t(u, wm2[...]) + bm2[...]
    return _dot_t(msg, wa[...]) + ba[...]


def _body(x_ref, win_ref, bin_ref, mem_ref,
          l0_wih, l0_whh_t, l0_bih, l0_bhh, l0_wm1_t, l0_bm1, l0_wm2,
          l0_bm2, l0_wa, l0_ba,
          l1_wih, l1_whh_t, l1_bih, l1_bhh, l1_wm1_t, l1_bm1, l1_wm2,
          l1_bm2, l1_wa, l1_ba,
          wc1_ref, bc1_ref, wc2_ref, bc2_ref, out_ref,
          h_scr, v_scr, sum_scr, mem_scr):
    s = pl.program_id(0)
    b = pl.program_id(1)
    rows = pl.ds(b * _BN, _BN)

    @pl.when(s == 0)
    def _stage0():
        hb = _dot_t(x_ref[...], win_ref[...]) + bin_ref[...]
        h_scr[rows, :] = hb
        psum = jnp.sum(hb, axis=0, keepdims=True)

        @pl.when(b == 0)
        def _():
            sum_scr[...] = psum

        @pl.when(b > 0)
        def _():
            sum_scr[...] += psum

    @pl.when(s == 1)
    def _stage1():
        @pl.when(b == 0)
        def _():
            xm = sum_scr[...] * (1.0 / _N)
            mem_scr[...] = _gru(mem_ref[...], xm,
                                l0_wih, l0_whh_t, l0_bih, l0_bhh)
            sum_scr[...] = jnp.zeros_like(sum_scr)

        nh = _mlp(h_scr[rows, :], mem_scr[...],
                  l0_wm1_t, l0_bm1, l0_wm2, l0_bm2, l0_wa, l0_ba)
        h_scr[rows, :] = nh
        sum_scr[...] += jnp.sum(nh, axis=0, keepdims=True)

    @pl.when(s == 2)
    def _stage2():
        @pl.when(b == 0)
        def _():
            xm = sum_scr[...] * (1.0 / _N)
            mem_scr[...] = _gru(mem_scr[...], xm,
                                l1_wih, l1_whh_t, l1_bih, l1_bhh)

        nh = _mlp(h_scr[rows, :], mem_scr[...],
                  l1_wm1_t, l1_bm1, l1_wm2, l1_bm2, l1_wa, l1_ba)
        v_scr[rows, :] = jnp.maximum(_dot_t(nh, wc1_ref[...])
                                     + bc1_ref[...], 0.0)

        @pl.when(b == _NB - 1)
        def _():
            out_ref[...] = (_dot_t(wc2_ref[...], v_scr[...])
                            + jnp.expand_dims(bc2_ref[...], 1))


def _full(arr_shape):
    nd = len(arr_shape)
    return pl.BlockSpec(arr_shape, lambda s, b: (0,) * nd)


def kernel(x, edge_index, edge_attr, t, W_in, b_in, memory,
           l0_wih, l0_whh, l0_bih, l0_bhh, l0_Wm1, l0_bm1, l0_Wm2, l0_bm2,
           l0_Wa, l0_ba,
           l1_wih, l1_whh, l1_bih, l1_bhh, l1_Wm1, l1_bm1, l1_Wm2, l1_bm2,
           l1_Wa, l1_ba,
           Wc1, bc1, Wc2, bc2):
    del edge_index, edge_attr, t
    f32 = jnp.float32
    args = (x, W_in, b_in, memory,
            l0_wih, l0_whh.T, l0_bih, l0_bhh, l0_Wm1.T, l0_bm1, l0_Wm2,
            l0_bm2, l0_Wa, l0_ba,
            l1_wih, l1_whh.T, l1_bih, l1_bhh, l1_Wm1.T, l1_bm1, l1_Wm2,
            l1_bm2, l1_Wa, l1_ba,
            Wc1, bc1, Wc2, bc2)
    x_spec = pl.BlockSpec(
        (_BN, _H), lambda s, b: (jnp.where(s == 0, b, _NB - 1), 0))
    in_specs = [x_spec] + [_full(a.shape) for a in args[1:]]
    out_t = pl.pallas_call(
        _body,
        grid=(3, _NB),
        in_specs=in_specs,
        out_specs=_full((2, _N)),
        out_shape=jax.ShapeDtypeStruct((2, _N), f32),
        scratch_shapes=[
            pltpu.VMEM((_N, _H), f32),
            pltpu.VMEM((_N, _M), f32),
            pltpu.VMEM((1, _H), f32),
            pltpu.VMEM((1, _M), f32),
        ],
    )(*args)
    return out_t.T


# speedup vs baseline: 2.3258x; 2.3258x over previous
import jax
import jax.numpy as jnp
from jax import lax
from jax.experimental import pallas as pl

_N = 10000
_H = 128
_M = 64

_DN_T = (((1,), (1,)), ((), ()))


def _dot_t(a, b):
    return lax.dot_general(a, b, _DN_T, preferred_element_type=jnp.float32)


def _dot(a, b):
    return jnp.dot(a, b, preferred_element_type=jnp.float32)


def _fused_body(x_ref, win_ref, bin_ref, mem_ref,
                l0_wih, l0_whh_t, l0_bih, l0_bhh, l0_wm1_t, l0_bm1, l0_wm2,
                l0_bm2, l0_wa, l0_ba,
                l1_wih, l1_whh_t, l1_bih, l1_bhh, l1_wm1_t, l1_bm1, l1_wm2,
                l1_bm2, l1_wa, l1_ba,
                wc1_ref, bc1_ref, wc2_ref, bc2_ref, out_ref):
    h = _dot_t(x_ref[...], win_ref[...]) + bin_ref[...]
    mem = mem_ref[...]
    for (wih, whh_t, bih, bhh, wm1_t, bm1, wm2, bm2, wa, ba) in (
            (l0_wih, l0_whh_t, l0_bih, l0_bhh, l0_wm1_t, l0_bm1, l0_wm2,
             l0_bm2, l0_wa, l0_ba),
            (l1_wih, l1_whh_t, l1_bih, l1_bhh, l1_wm1_t, l1_bm1, l1_wm2,
             l1_bm2, l1_wa, l1_ba)):
        xm = jnp.sum(h, axis=0, keepdims=True) * (1.0 / _N)
        gi_r = _dot_t(xm, wih[0:_M, :]) + bih[0:_M]
        gi_z = _dot_t(xm, wih[_M:2 * _M, :]) + bih[_M:2 * _M]
        gi_n = _dot_t(xm, wih[2 * _M:, :]) + bih[2 * _M:]
        gh = _dot(mem, whh_t[...]) + bhh[...]
        r = jax.nn.sigmoid(gi_r + gh[:, 0:_M])
        z = jax.nn.sigmoid(gi_z + gh[:, _M:2 * _M])
        n = jnp.tanh(gi_n + r * gh[:, 2 * _M:])
        mem = (1.0 - z) * n + z * mem
        c = _dot(mem, wm1_t[_H:, :]) + bm1[...]
        u = jnp.maximum(_dot(h, wm1_t[0:_H, :]) + c, 0.0)
        msg = _dot_t(u, wm2[...]) + bm2[...]
        h = _dot_t(msg, wa[...]) + ba[...]
    v = jnp.maximum(_dot_t(h, wc1_ref[...]) + bc1_ref[...], 0.0)
    out_ref[...] = (_dot_t(wc2_ref[...], v)
                    + jnp.expand_dims(bc2_ref[...], 1))


def kernel(x, edge_index, edge_attr, t, W_in, b_in, memory,
           l0_wih, l0_whh, l0_bih, l0_bhh, l0_Wm1, l0_bm1, l0_Wm2, l0_bm2,
           l0_Wa, l0_ba,
           l1_wih, l1_whh, l1_bih, l1_bhh, l1_Wm1, l1_bm1, l1_Wm2, l1_bm2,
           l1_Wa, l1_ba,
           Wc1, bc1, Wc2, bc2):
    del edge_index, edge_attr, t
    out_t = pl.pallas_call(
        _fused_body,
        out_shape=jax.ShapeDtypeStruct((2, _N), jnp.float32),
    )(x, W_in, b_in, memory,
      l0_wih, l0_whh.T, l0_bih, l0_bhh, l0_Wm1.T, l0_bm1, l0_Wm2, l0_bm2,
      l0_Wa, l0_ba,
      l1_wih, l1_whh.T, l1_bih, l1_bhh, l1_Wm1.T, l1_bm1, l1_Wm2, l1_bm2,
      l1_Wa, l1_ba,
      Wc1, bc1, Wc2, bc2)
    return out_t.T
